# Initial kernel scaffold; baseline (speedup 1.0000x reference)
import jax
import jax.numpy as jnp
from jax.experimental import pallas as pl


def kernel(x, boundaries, tables):
    raise NotImplementedError("write your pallas kernel here")


# speedup vs baseline: 67.9981x; 67.9981x over previous
import functools

import jax
import jax.numpy as jnp
from jax import lax
from jax.experimental import pallas as pl
from jax.experimental.pallas import tpu as pltpu
from jax.experimental.pallas import tpu_sc as plsc

BATCH = 16384
NF = 100
NBND = 99
BPAD = 128
NROWS = NF * 101
EMB = 128

NC, NS, L = 2, 16, 16
NW = NC * NS
SPAN = BATCH // NW
BSUB = 64
MID = SPAN // BSUB
ESUB = BSUB * NF
NVEC = ESUB // L
G = 128
NG = ESUB // G


def _sc_body(xf, bnd, tbl, out, x_v, b_v, idx_v, rows_v, gsem):
    wid = lax.axis_index("s") * NC + lax.axis_index("c")
    e0w = wid * (SPAN * NF)

    pltpu.sync_copy(bnd, b_v)

    iota = lax.iota(jnp.int32, L)

    for m in range(MID):
        e0 = e0w + m * ESUB
        pltpu.sync_copy(xf.at[pl.ds(e0, ESUB)], x_v)

        def compute(v, carry, e0=e0):
            base = v * L
            xv = x_v[pl.ds(base, L)]
            f = lax.rem(e0 + base + iota, NF)
            fb = f * BPAD
            lo = jnp.zeros((L,), jnp.int32)
            for p in (64, 32, 16, 8, 4, 2, 1):
                cand = lo + p
                probe = plsc.load_gather(b_v, [fb + cand - 1])
                lo = jnp.where(probe < xv, cand, lo)
            tidx = f * 101 + lo
            idx_v[v // (G // L), pl.ds(lax.rem(v, G // L) * L, L)] = tidx
            return carry

        lax.fori_loop(0, NVEC, compute, 0)

        def move(j, carry, e0=e0):
            pltpu.async_copy(tbl.at[idx_v.at[j]], rows_v, gsem).wait()
            pltpu.sync_copy(rows_v, out.at[pl.ds(e0 + j * G, G)])
            return carry

        lax.fori_loop(0, NG, move, 0)


def kernel(x, boundaries, tables):
    xf = x.reshape(BATCH * NF)
    bnd = jnp.concatenate(
        [boundaries, jnp.full((NF, BPAD - NBND), jnp.inf, jnp.float32)], axis=1
    ).reshape(NF * BPAD)
    tbl = tables.reshape(NROWS, EMB)

    mesh = plsc.VectorSubcoreMesh(core_axis_name="c", subcore_axis_name="s")
    run = functools.partial(
        pl.kernel,
        mesh=mesh,
        out_type=jax.ShapeDtypeStruct((BATCH * NF, EMB), jnp.float32),
        scratch_types=[
            pltpu.VMEM((ESUB,), jnp.float32),
            pltpu.VMEM((NF * BPAD,), jnp.float32),
            pltpu.VMEM((NG, G), jnp.int32),
            pltpu.VMEM((G, EMB), jnp.float32),
            pltpu.SemaphoreType.DMA,
        ],
        compiler_params=pltpu.CompilerParams(needs_layout_passes=False),
    )(_sc_body)
    out = run(xf, bnd, tbl)
    return out.reshape(BATCH, NF * EMB)


# speedup vs baseline: 76.6059x; 1.1266x over previous
import functools

import jax
import jax.numpy as jnp
from jax import lax
from jax.experimental import pallas as pl
from jax.experimental.pallas import tpu as pltpu
from jax.experimental.pallas import tpu_sc as plsc

BATCH = 16384
NF = 100
NBND = 99
BPAD = 128
NROWS = NF * 101
EMB = 128

NC, NS, L = 2, 16, 16
NW = NC * NS
SPAN = BATCH // NW
BSUB = 64
MID = SPAN // BSUB
ESUB = BSUB * NF
NVEC = ESUB // L
G = 128
NG = ESUB // G


def _sc_body(xf, bnd, tbl, out, x_v, b_v, idx_v, rows0, rows1, gs0, gs1, ws0, ws1):
    wid = lax.axis_index("s") * NC + lax.axis_index("c")
    e0w = wid * (SPAN * NF)

    pltpu.sync_copy(bnd, b_v)

    iota = lax.iota(jnp.int32, L)

    for m in range(MID):
        e0 = e0w + m * ESUB
        pltpu.sync_copy(xf.at[pl.ds(e0, ESUB)], x_v)

        def compute(v, carry, e0=e0):
            base = v * L
            xv = x_v[pl.ds(base, L)]
            f = lax.rem(e0 + base + iota, NF)
            fb = f * BPAD
            lo = jnp.zeros((L,), jnp.int32)
            for p in (64, 32, 16, 8, 4, 2, 1):
                cand = lo + p
                probe = plsc.load_gather(b_v, [fb + cand - 1])
                lo = jnp.where(probe < xv, cand, lo)
            tidx = f * 101 + lo
            idx_v[v // (G // L), pl.ds(lax.rem(v, G // L) * L, L)] = tidx
            return carry

        lax.fori_loop(0, NVEC, compute, 0)

        pltpu.async_copy(tbl.at[idx_v.at[0]], rows0, gs0)
        pltpu.async_copy(tbl.at[idx_v.at[1]], rows1, gs1)

        def move(jj, carry, e0=e0):
            j0 = 2 * jj
            pltpu.make_async_copy(tbl.at[idx_v.at[j0]], rows0, gs0).wait()
            pltpu.async_copy(rows0, out.at[pl.ds(e0 + j0 * G, G)], ws0)
            pltpu.make_async_copy(tbl.at[idx_v.at[j0 + 1]], rows1, gs1).wait()
            pltpu.async_copy(rows1, out.at[pl.ds(e0 + (j0 + 1) * G, G)], ws1)
            pltpu.make_async_copy(rows0, out.at[pl.ds(e0 + j0 * G, G)], ws0).wait()

            @pl.when(jj < NG // 2 - 1)
            def _():
                pltpu.async_copy(tbl.at[idx_v.at[j0 + 2]], rows0, gs0)

            pltpu.make_async_copy(
                rows1, out.at[pl.ds(e0 + (j0 + 1) * G, G)], ws1
            ).wait()

            @pl.when(jj < NG // 2 - 1)
            def _():
                pltpu.async_copy(tbl.at[idx_v.at[j0 + 3]], rows1, gs1)

            return carry

        lax.fori_loop(0, NG // 2, move, 0)


def kernel(x, boundaries, tables):
    xf = x.reshape(BATCH * NF)
    bnd = jnp.concatenate(
        [boundaries, jnp.full((NF, BPAD - NBND), jnp.inf, jnp.float32)], axis=1
    ).reshape(NF * BPAD)
    tbl = tables.reshape(NROWS, EMB)

    mesh = plsc.VectorSubcoreMesh(core_axis_name="c", subcore_axis_name="s")
    run = functools.partial(
        pl.kernel,
        mesh=mesh,
        out_type=jax.ShapeDtypeStruct((BATCH * NF, EMB), jnp.float32),
        scratch_types=[
            pltpu.VMEM((ESUB,), jnp.float32),
            pltpu.VMEM((NF * BPAD,), jnp.float32),
            pltpu.VMEM((NG, G), jnp.int32),
            pltpu.VMEM((G, EMB), jnp.float32),
            pltpu.VMEM((G, EMB), jnp.float32),
            pltpu.SemaphoreType.DMA,
            pltpu.SemaphoreType.DMA,
            pltpu.SemaphoreType.DMA,
            pltpu.SemaphoreType.DMA,
        ],
        compiler_params=pltpu.CompilerParams(needs_layout_passes=False),
    )(_sc_body)
    out = run(xf, bnd, tbl)
    return out.reshape(BATCH, NF * EMB)


# speedup vs baseline: 82.7642x; 1.0804x over previous
import functools

import jax
import jax.numpy as jnp
from jax import lax
from jax.experimental import pallas as pl
from jax.experimental.pallas import tpu as pltpu
from jax.experimental.pallas import tpu_sc as plsc

BATCH = 16384
NF = 100
NBND = 99
BPAD = 128
NROWS = NF * 101
EMB = 128

NC, NS, L = 2, 16, 16
NW = NC * NS
SPAN = BATCH // NW
BSUB = 64
MID = SPAN // BSUB
ESUB = BSUB * NF
NVEC = ESUB // L
G = 64
NG = ESUB // G


def _sc_body(
    xf, bnd, tbl, out, x_v, b_v, idx_v, rows0, rows1, tbl_sh, gs0, gs1, ws0, ws1
):
    wid = lax.axis_index("s") * NC + lax.axis_index("c")
    e0w = wid * (SPAN * NF)

    @pl.when(lax.axis_index("s") == 0)
    def _():
        pltpu.sync_copy(tbl, tbl_sh)

    pltpu.sync_copy(bnd, b_v)
    plsc.subcore_barrier()

    iota = lax.iota(jnp.int32, L)

    for m in range(MID):
        e0 = e0w + m * ESUB
        pltpu.sync_copy(xf.at[pl.ds(e0, ESUB)], x_v)

        def compute(v, carry, e0=e0):
            base = v * L
            xv = x_v[pl.ds(base, L)]
            f = lax.rem(e0 + base + iota, NF)
            fb = f * BPAD
            lo = jnp.zeros((L,), jnp.int32)
            for p in (64, 32, 16, 8, 4, 2, 1):
                cand = lo + p
                probe = plsc.load_gather(b_v, [fb + cand - 1])
                lo = jnp.where(probe < xv, cand, lo)
            tidx = f * 101 + lo
            idx_v[v // (G // L), pl.ds(lax.rem(v, G // L) * L, L)] = tidx
            return carry

        lax.fori_loop(0, NVEC, compute, 0)

        pltpu.async_copy(tbl_sh.at[idx_v.at[0]], rows0, gs0)
        pltpu.async_copy(tbl_sh.at[idx_v.at[1]], rows1, gs1)

        def move(jj, carry, e0=e0):
            j0 = 2 * jj
            pltpu.make_async_copy(tbl_sh.at[idx_v.at[j0]], rows0, gs0).wait()
            pltpu.async_copy(rows0, out.at[pl.ds(e0 + j0 * G, G)], ws0)
            pltpu.make_async_copy(tbl_sh.at[idx_v.at[j0 + 1]], rows1, gs1).wait()
            pltpu.async_copy(rows1, out.at[pl.ds(e0 + (j0 + 1) * G, G)], ws1)
            pltpu.make_async_copy(rows0, out.at[pl.ds(e0 + j0 * G, G)], ws0).wait()

            @pl.when(jj < NG // 2 - 1)
            def _():
                pltpu.async_copy(tbl_sh.at[idx_v.at[j0 + 2]], rows0, gs0)

            pltpu.make_async_copy(
                rows1, out.at[pl.ds(e0 + (j0 + 1) * G, G)], ws1
            ).wait()

            @pl.when(jj < NG // 2 - 1)
            def _():
                pltpu.async_copy(tbl_sh.at[idx_v.at[j0 + 3]], rows1, gs1)

            return carry

        lax.fori_loop(0, NG // 2, move, 0)


def kernel(x, boundaries, tables):
    xf = x.reshape(BATCH * NF)
    bnd = jnp.concatenate(
        [boundaries, jnp.full((NF, BPAD - NBND), jnp.inf, jnp.float32)], axis=1
    ).reshape(NF * BPAD)
    tbl = tables.reshape(NROWS, EMB)

    mesh = plsc.VectorSubcoreMesh(core_axis_name="c", subcore_axis_name="s")
    run = functools.partial(
        pl.kernel,
        mesh=mesh,
        out_type=jax.ShapeDtypeStruct((BATCH * NF, EMB), jnp.float32),
        scratch_types=[
            pltpu.VMEM((ESUB,), jnp.float32),
            pltpu.VMEM((NF * BPAD,), jnp.float32),
            pltpu.VMEM((NG, G), jnp.int32),
            pltpu.VMEM((G, EMB), jnp.float32),
            pltpu.VMEM((G, EMB), jnp.float32),
            pltpu.VMEM_SHARED((NROWS, EMB), jnp.float32),
            pltpu.SemaphoreType.DMA,
            pltpu.SemaphoreType.DMA,
            pltpu.SemaphoreType.DMA,
            pltpu.SemaphoreType.DMA,
        ],
        compiler_params=pltpu.CompilerParams(needs_layout_passes=False),
    )(_sc_body)
    out = run(xf, bnd, tbl)
    return out.reshape(BATCH, NF * EMB)


# speedup vs baseline: 93.9758x; 1.1355x over previous
import functools

import jax
import jax.numpy as jnp
from jax import lax
from jax.experimental import pallas as pl
from jax.experimental.pallas import tpu as pltpu
from jax.experimental.pallas import tpu_sc as plsc

BATCH = 16384
NF = 100
NBND = 99
BPAD = 128
NROWS = NF * 101
EMB = 128

NC, NS, L = 2, 16, 16
NW = NC * NS
SPAN = BATCH // NW
BSUB = 32
MID = SPAN // BSUB
ESUB = BSUB * NF
NVEC = ESUB // L
G = 32
NG = ESUB // G
NBUF = 4
NRND = NG // NBUF


def _sc_body(xf, bnd, tbl, out, x_v, b_v, idx_v, rows, tbl_sh, gsems, wsems):
    wid = lax.axis_index("s") * NC + lax.axis_index("c")
    e0w = wid * (SPAN * NF)

    @pl.when(lax.axis_index("s") == 0)
    def _():
        pltpu.sync_copy(tbl, tbl_sh)

    pltpu.sync_copy(bnd, b_v)
    plsc.subcore_barrier()

    iota = lax.iota(jnp.int32, L)

    def mid(m, carry_m):
        e0 = e0w + m * ESUB
        pltpu.sync_copy(xf.at[pl.ds(e0, ESUB)], x_v)

        def compute(v, carry, e0=e0):
            base = v * L
            xv = x_v[pl.ds(base, L)]
            f = lax.rem(e0 + base + iota, NF)
            fb = f * BPAD
            lo = jnp.zeros((L,), jnp.int32)
            for p in (64, 32, 16, 8, 4, 2, 1):
                cand = lo + p
                probe = plsc.load_gather(b_v, [fb + cand - 1])
                lo = jnp.where(probe < xv, cand, lo)
            tidx = f * 101 + lo
            idx_v[v // (G // L), pl.ds(lax.rem(v, G // L) * L, L)] = tidx
            return carry

        lax.fori_loop(0, NVEC, compute, 0)

        for b in range(NBUF):
            pltpu.async_copy(tbl_sh.at[idx_v.at[b]], rows.at[b], gsems.at[b])

        def move(jj, carry, e0=e0):
            j0 = jj * NBUF
            for b in range(NBUF):
                pltpu.make_async_copy(
                    tbl_sh.at[idx_v.at[j0 + b]], rows.at[b], gsems.at[b]
                ).wait()
                pltpu.async_copy(
                    rows.at[b], out.at[pl.ds(e0 + (j0 + b) * G, G)], wsems.at[b]
                )
            for b in range(NBUF):
                pltpu.make_async_copy(
                    rows.at[b], out.at[pl.ds(e0 + (j0 + b) * G, G)], wsems.at[b]
                ).wait()

                @pl.when(jj < NRND - 1)
                def _(b=b):
                    pltpu.async_copy(
                        tbl_sh.at[idx_v.at[j0 + NBUF + b]], rows.at[b], gsems.at[b]
                    )

            return carry

        lax.fori_loop(0, NRND, move, 0)
        return carry_m

    lax.fori_loop(0, MID, mid, 0)


def kernel(x, boundaries, tables):
    xf = x.reshape(BATCH * NF)
    bnd = jnp.concatenate(
        [boundaries, jnp.full((NF, BPAD - NBND), jnp.inf, jnp.float32)], axis=1
    ).reshape(NF * BPAD)
    tbl = tables.reshape(NROWS, EMB)

    mesh = plsc.VectorSubcoreMesh(core_axis_name="c", subcore_axis_name="s")
    run = functools.partial(
        pl.kernel,
        mesh=mesh,
        out_type=jax.ShapeDtypeStruct((BATCH * NF, EMB), jnp.float32),
        scratch_types=[
            pltpu.VMEM((ESUB,), jnp.float32),
            pltpu.VMEM((NF * BPAD,), jnp.float32),
            pltpu.VMEM((NG, G), jnp.int32),
            pltpu.VMEM((NBUF, G, EMB), jnp.float32),
            pltpu.VMEM_SHARED((NROWS, EMB), jnp.float32),
            pltpu.SemaphoreType.DMA((NBUF,)),
            pltpu.SemaphoreType.DMA((NBUF,)),
        ],
        compiler_params=pltpu.CompilerParams(needs_layout_passes=False),
    )(_sc_body)
    out = run(xf, bnd, tbl)
    return out.reshape(BATCH, NF * EMB)
